# Initial kernel scaffold; baseline (speedup 1.0000x reference)
import jax
import jax.numpy as jnp
from jax.experimental import pallas as pl


def kernel(x, theta, w_out, b_out):
    raise NotImplementedError("write your pallas kernel here")


# speedup vs baseline: 2.0286x; 2.0286x over previous
import math

import jax
import jax.numpy as jnp
import numpy as np
from jax.experimental import pallas as pl
from jax.experimental.pallas import tpu as pltpu

B, S, E = 4, 2048, 1024
H, DK = 16, 64
SCALE = 1.0 / math.sqrt(DK)

_C1 = float(np.uint32(0x3FC90FDA).view(np.float32))
_C2 = float(np.uint32(0x33A22168).view(np.float32))
_TWO_OVER_PI = 0.6366197723675814
_RND = 12582912.0


def _fast_cos(a):
    t = a * _TWO_OVER_PI
    ki = jnp.round(t).astype(jnp.int32)
    k = ki.astype(jnp.float32)
    r = a - k * _C1
    r = r - k * _C2
    r2 = r * r
    cosr = 1.0 + r2 * (-0.5 + r2 * (1.0 / 24.0 + r2 * (-1.0 / 720.0)))
    sinr = r * (1.0 + r2 * (-1.0 / 6.0 + r2 * (1.0 / 120.0 + r2 * (-1.0 / 5040.0))))
    val = jnp.where((ki & 1) == 0, cosr, sinr)
    return jnp.where(((ki + 1) & 2) == 0, val, -val)


def _cos_body(x_ref, th_ref, o_ref):
    ang = x_ref[0] + th_ref[0]
    o_ref[0] = _fast_cos(ang).astype(jnp.bfloat16)


def _attn_body(kv_ref, o_ref, kvx_ref):
    lane = jax.lax.broadcasted_iota(jnp.int32, (S, 128 - DK), 1)
    kvx_ref[:, :DK] = kv_ref[0]
    kvx_ref[:, DK:] = jnp.where(lane == 0, 1.0, 0.0).astype(jnp.bfloat16)
    kvx = kvx_ref[:, :]

    qb, kb = 256, 512
    for qi in range(S // qb):
        q = kvx_ref[qi * qb:(qi + 1) * qb, :] * jnp.bfloat16(SCALE)
        acc = jnp.zeros((qb, 128), jnp.float32)
        for ki in range(S // kb):
            kc = kvx[ki * kb:(ki + 1) * kb, :]
            s = jax.lax.dot_general(
                q, kc, (((1,), (1,)), ((), ())),
                preferred_element_type=jnp.float32)
            e = jnp.exp(s).astype(jnp.bfloat16)
            acc = acc + jax.lax.dot_general(
                e, kc, (((1,), (0,)), ((), ())),
                preferred_element_type=jnp.float32)
        recip = 1.0 / acc[:, DK:DK + 1]
        o_ref[0, qi * qb:(qi + 1) * qb, :] = (
            acc[:, :DK] * recip).astype(jnp.bfloat16)


def _proj_body(a_ref, w_ref, b_ref, o_ref):
    o_ref[...] = (
        jnp.dot(a_ref[...], w_ref[...], preferred_element_type=jnp.float32)
        + b_ref[...])


@jax.jit
def kernel(x, theta, w_out, b_out):
    sb = 512
    qkv = pl.pallas_call(
        _cos_body,
        grid=(B, S // sb),
        in_specs=[
            pl.BlockSpec((1, sb, E), lambda b, i: (b, i, 0)),
            pl.BlockSpec((1, 1, E), lambda b, i: (0, 0, 0)),
        ],
        out_specs=pl.BlockSpec((1, sb, E), lambda b, i: (b, i, 0)),
        out_shape=jax.ShapeDtypeStruct((B, S, E), jnp.bfloat16),
        compiler_params=pltpu.CompilerParams(
            dimension_semantics=("parallel", "parallel")),
    )(x, theta.reshape(1, 1, E))

    qkv_t = qkv.reshape(B, S, H, DK).transpose(0, 2, 1, 3).reshape(B * H, S, DK)

    att = pl.pallas_call(
        _attn_body,
        grid=(B * H,),
        in_specs=[pl.BlockSpec((1, S, DK), lambda i: (i, 0, 0))],
        out_specs=pl.BlockSpec((1, S, DK), lambda i: (i, 0, 0)),
        out_shape=jax.ShapeDtypeStruct((B * H, S, DK), jnp.bfloat16),
        scratch_shapes=[pltpu.VMEM((S, 128), jnp.bfloat16)],
        compiler_params=pltpu.CompilerParams(
            dimension_semantics=("parallel",)),
    )(qkv_t)

    a2d = att.reshape(B, H, S, DK).transpose(0, 2, 1, 3).reshape(B * S, E)
    wt = w_out.T.astype(jnp.bfloat16)

    mb = 1024
    y = pl.pallas_call(
        _proj_body,
        grid=(B * S // mb,),
        in_specs=[
            pl.BlockSpec((mb, E), lambda i: (i, 0)),
            pl.BlockSpec((E, E), lambda i: (0, 0)),
            pl.BlockSpec((1, E), lambda i: (0, 0)),
        ],
        out_specs=pl.BlockSpec((mb, E), lambda i: (i, 0)),
        out_shape=jax.ShapeDtypeStruct((B * S, E), jnp.float32),
        compiler_params=pltpu.CompilerParams(
            dimension_semantics=("parallel",)),
    )(a2d, wt, b_out.reshape(1, E))

    return y.reshape(B, S, E)


# speedup vs baseline: 2.3970x; 1.1816x over previous
import math

import jax
import jax.numpy as jnp
import numpy as np
from jax.experimental import pallas as pl
from jax.experimental.pallas import tpu as pltpu

B, S, E = 4, 2048, 1024
H, DK = 16, 64
SCALE = 1.0 / math.sqrt(DK)

_C1 = float(np.uint32(0x3FC90FDA).view(np.float32))
_C2 = float(np.uint32(0x33A22168).view(np.float32))
_TWO_OVER_PI = 0.6366197723675814


def _fast_cos(a):
    t = a * _TWO_OVER_PI
    ki = jnp.round(t).astype(jnp.int32)
    k = ki.astype(jnp.float32)
    r = a - k * _C1
    r = r - k * _C2
    r2 = r * r
    cosr = 1.0 + r2 * (-0.5 + r2 * (1.0 / 24.0 + r2 * (-1.0 / 720.0)))
    sinr = r * (1.0 + r2 * (-1.0 / 6.0 + r2 * (1.0 / 120.0 + r2 * (-1.0 / 5040.0))))
    val = jnp.where((ki & 1) == 0, cosr, sinr)
    return jnp.where(((ki + 1) & 2) == 0, val, -val)


def _cos_body(x_ref, th_ref, o_ref):
    ang = x_ref[0] + th_ref[0]
    c = _fast_cos(ang).astype(jnp.bfloat16)
    for h in range(H):
        o_ref[0, h] = c[:, h * DK:(h + 1) * DK]


def _attn_body(kv_ref, o_ref, kvx_ref):
    lane = jax.lax.broadcasted_iota(jnp.int32, (S, 128 - DK), 1)
    kvx_ref[:, :DK] = kv_ref[0]
    kvx_ref[:, DK:] = jnp.where(lane == 0, 1.0, 0.0).astype(jnp.bfloat16)
    kvx = kvx_ref[:, :]

    qb, kb = 256, 512
    for qi in range(S // qb):
        q = kvx_ref[qi * qb:(qi + 1) * qb, :] * jnp.bfloat16(SCALE)
        acc = jnp.zeros((qb, 128), jnp.float32)
        for ki in range(S // kb):
            kc = kvx[ki * kb:(ki + 1) * kb, :]
            s = jax.lax.dot_general(
                q, kc, (((1,), (1,)), ((), ())),
                preferred_element_type=jnp.float32)
            e = jnp.exp(s).astype(jnp.bfloat16)
            acc = acc + jax.lax.dot_general(
                e, kc, (((1,), (0,)), ((), ())),
                preferred_element_type=jnp.float32)
        recip = 1.0 / acc[:, DK:DK + 1]
        o_ref[0, qi * qb:(qi + 1) * qb, :] = (
            acc[:, :DK] * recip).astype(jnp.bfloat16)


def _proj_body(a_ref, w_ref, b_ref, o_ref):
    a = jnp.concatenate([a_ref[0, h] for h in range(H)], axis=-1)
    o_ref[0] = (
        jnp.dot(a, w_ref[...], preferred_element_type=jnp.float32)
        + b_ref[...])


@jax.jit
def kernel(x, theta, w_out, b_out):
    sb = 512
    qkv_t = pl.pallas_call(
        _cos_body,
        grid=(B, S // sb),
        in_specs=[
            pl.BlockSpec((1, sb, E), lambda b, i: (b, i, 0)),
            pl.BlockSpec((1, 1, E), lambda b, i: (0, 0, 0)),
        ],
        out_specs=pl.BlockSpec((1, H, sb, DK), lambda b, i: (b, 0, i, 0)),
        out_shape=jax.ShapeDtypeStruct((B, H, S, DK), jnp.bfloat16),
        compiler_params=pltpu.CompilerParams(
            dimension_semantics=("parallel", "parallel")),
    )(x, theta.reshape(1, 1, E))

    qkv_f = qkv_t.reshape(B * H, S, DK)

    att = pl.pallas_call(
        _attn_body,
        grid=(B * H,),
        in_specs=[pl.BlockSpec((1, S, DK), lambda i: (i, 0, 0))],
        out_specs=pl.BlockSpec((1, S, DK), lambda i: (i, 0, 0)),
        out_shape=jax.ShapeDtypeStruct((B * H, S, DK), jnp.bfloat16),
        scratch_shapes=[pltpu.VMEM((S, 128), jnp.bfloat16)],
        compiler_params=pltpu.CompilerParams(
            dimension_semantics=("parallel",)),
    )(qkv_f)

    att4 = att.reshape(B, H, S, DK)
    wt = w_out.T.astype(jnp.bfloat16)

    mb = 512
    y = pl.pallas_call(
        _proj_body,
        grid=(B, S // mb),
        in_specs=[
            pl.BlockSpec((1, H, mb, DK), lambda b, i: (b, 0, i, 0)),
            pl.BlockSpec((E, E), lambda b, i: (0, 0)),
            pl.BlockSpec((1, E), lambda b, i: (0, 0)),
        ],
        out_specs=pl.BlockSpec((1, mb, E), lambda b, i: (b, i, 0)),
        out_shape=jax.ShapeDtypeStruct((B, S, E), jnp.float32),
        compiler_params=pltpu.CompilerParams(
            dimension_semantics=("parallel", "parallel")),
    )(att4, wt, b_out.reshape(1, E))

    return y


# speedup vs baseline: 3.7662x; 1.5712x over previous
import math

import jax
import jax.numpy as jnp
import numpy as np
from jax.experimental import pallas as pl
from jax.experimental.pallas import tpu as pltpu

B, S, E = 4, 2048, 1024
H, DK = 16, 64
SCALE = 1.0 / math.sqrt(DK)

_C1 = float(np.uint32(0x3FC90FDA).view(np.float32))
_C2 = float(np.uint32(0x33A22168).view(np.float32))
_TWO_OVER_PI = 0.6366197723675814
_LOG2E = 1.4426950408889634


def _fast_cos(a):
    t = a * _TWO_OVER_PI
    ki = jnp.round(t).astype(jnp.int32)
    k = ki.astype(jnp.float32)
    r = a - k * _C1
    r = r - k * _C2
    r2 = r * r
    cosr = 1.0 + r2 * (-0.5 + r2 * (1.0 / 24.0 + r2 * (-1.0 / 720.0)))
    sinr = r * (1.0 + r2 * (-1.0 / 6.0 + r2 * (1.0 / 120.0 + r2 * (-1.0 / 5040.0))))
    val = jnp.where((ki & 1) == 0, cosr, sinr)
    return jnp.where(((ki + 1) & 2) == 0, val, -val)


def _cos_body(x_ref, th_ref, o_ref):
    ang = x_ref[0] + th_ref[0]
    c = _fast_cos(ang).astype(jnp.bfloat16)
    for h in range(H):
        o_ref[0, h] = c[:, h * DK:(h + 1) * DK]


def _attn_body(kv_ref, o_ref, kvx_ref, e_ref):
    lane = jax.lax.broadcasted_iota(jnp.int32, (S, 128 - DK), 1)
    kvx_ref[:, :DK] = kv_ref[0]
    kvx_ref[:, DK:] = jnp.where(lane == 0, 1.0, 0.0).astype(jnp.bfloat16)

    qb, kb = 256, 512
    for qi in range(S // qb):
        q = kvx_ref[qi * qb:(qi + 1) * qb, :] * jnp.bfloat16(SCALE * _LOG2E)
        for ki in range(S // kb):
            kc = kvx_ref[ki * kb:(ki + 1) * kb, :]
            s = jax.lax.dot_general(
                q, kc, (((1,), (1,)), ((), ())),
                preferred_element_type=jnp.float32)
            e_ref[:, ki * kb:(ki + 1) * kb] = jnp.exp2(s).astype(jnp.bfloat16)
        acc = jax.lax.dot_general(
            e_ref[:, :], kvx_ref[:, :], (((1,), (0,)), ((), ())),
            preferred_element_type=jnp.float32)
        recip = 1.0 / acc[:, DK:DK + 1]
        o_ref[0, qi * qb:(qi + 1) * qb, :] = (
            acc[:, :DK] * recip).astype(jnp.bfloat16)


def _proj_body(a_ref, w_ref, b_ref, o_ref):
    a = jnp.concatenate([a_ref[0, h] for h in range(H)], axis=-1)
    o_ref[0] = (
        jnp.dot(a, w_ref[...], preferred_element_type=jnp.float32)
        + b_ref[...])


@jax.jit
def kernel(x, theta, w_out, b_out):
    sb = 512
    nsb = S // sb
    qkv_t = pl.pallas_call(
        _cos_body,
        grid=(2, B * nsb // 2),
        in_specs=[
            pl.BlockSpec((1, sb, E),
                         lambda c, i: ((c * (B * nsb // 2) + i) // nsb,
                                       (c * (B * nsb // 2) + i) % nsb, 0)),
            pl.BlockSpec((1, 1, E), lambda c, i: (0, 0, 0)),
        ],
        out_specs=pl.BlockSpec(
            (1, H, sb, DK),
            lambda c, i: ((c * (B * nsb // 2) + i) // nsb, 0,
                          (c * (B * nsb // 2) + i) % nsb, 0)),
        out_shape=jax.ShapeDtypeStruct((B, H, S, DK), jnp.bfloat16),
        compiler_params=pltpu.CompilerParams(
            dimension_semantics=("parallel", "parallel")),
    )(x, theta.reshape(1, 1, E))

    qkv_f = qkv_t.reshape(B * H, S, DK)

    att = pl.pallas_call(
        _attn_body,
        grid=(2, B * H // 2),
        in_specs=[pl.BlockSpec((1, S, DK),
                               lambda c, i: (c * (B * H // 2) + i, 0, 0))],
        out_specs=pl.BlockSpec((1, S, DK),
                               lambda c, i: (c * (B * H // 2) + i, 0, 0)),
        out_shape=jax.ShapeDtypeStruct((B * H, S, DK), jnp.bfloat16),
        scratch_shapes=[pltpu.VMEM((S, 128), jnp.bfloat16),
                        pltpu.VMEM((256, S), jnp.bfloat16)],
        compiler_params=pltpu.CompilerParams(
            dimension_semantics=("parallel", "parallel")),
    )(qkv_f)

    att4 = att.reshape(B, H, S, DK)
    wt = w_out.T.astype(jnp.bfloat16)

    mb = 512
    nmb = S // mb
    y = pl.pallas_call(
        _proj_body,
        grid=(2, B * nmb // 2),
        in_specs=[
            pl.BlockSpec((1, H, mb, DK),
                         lambda c, i: ((c * (B * nmb // 2) + i) // nmb, 0,
                                       (c * (B * nmb // 2) + i) % nmb, 0)),
            pl.BlockSpec((E, E), lambda c, i: (0, 0)),
            pl.BlockSpec((1, E), lambda c, i: (0, 0)),
        ],
        out_specs=pl.BlockSpec(
            (1, mb, E),
            lambda c, i: ((c * (B * nmb // 2) + i) // nmb,
                          (c * (B * nmb // 2) + i) % nmb, 0)),
        out_shape=jax.ShapeDtypeStruct((B, S, E), jnp.float32),
        compiler_params=pltpu.CompilerParams(
            dimension_semantics=("parallel", "parallel")),
    )(att4, wt, b_out.reshape(1, E))

    return y
